# baseline (speedup 1.0000x reference)
import functools

import jax
import jax.numpy as jnp
from jax import lax
from jax.experimental import pallas as pl
from jax.experimental.pallas import tpu as pltpu
from jax.experimental.pallas import tpu_sc as plsc

NC, NS, L = 2, 16, 16
NW = NC * NS
R = 147456
NROWS = 384
ROWS_PER_W = NROWS // NW
K = 2949
CH = 24576
NCHUNK = R // CH
UNROLL = 8
NB1, NB2, NB3 = 2048, 2048, 1024
HIST_WORDS = NB1
CAP = 32768


def _sc_body(x_hbm, out_hbm, hist, buf0, buf1, cand, res, sem0, sem1):
    wid = lax.axis_index("s") * NC + lax.axis_index("c")
    lane = lax.iota(jnp.int32, L)
    lane_u = lax.iota(jnp.uint32, L)
    zero16i = jnp.zeros((L,), jnp.int32)
    zero16f = jnp.zeros((L,), jnp.float32)
    ones16i = jnp.full((L,), 1, jnp.int32)

    def monotone_key(xv):
        u = plsc.bitcast(xv, jnp.uint32)
        s = plsc.bitcast(
            lax.shift_right_arithmetic(plsc.bitcast(xv, jnp.int32), 31),
            jnp.uint32)
        return u ^ (s | jnp.uint32(0x80000000))

    def clear_hist():
        @plsc.parallel_loop(0, HIST_WORDS // L, unroll=8)
        def _(i):
            hist[pl.ds(i * L, L)] = zero16i

    def stream_pass(base, vec_fn, carry):
        def dma(c, buf, sem):
            return pltpu.make_async_copy(
                x_hbm.at[pl.ds(base + c * CH, CH)], buf, sem)

        dma(0, buf0, sem0).start()
        dma(1, buf1, sem1).start()

        def chunk(buf, carry):
            @plsc.parallel_loop(0, CH // L, unroll=UNROLL, carry=carry)
            def out(j, c):
                xv = buf[pl.ds(j * L, L)]
                return vec_fn(xv, c)
            return out

        def step(i, carry):
            dma(2 * i, buf0, sem0).wait()
            carry = chunk(buf0, carry)

            @pl.when(i < NCHUNK // 2 - 1)
            def _():
                dma(2 * i + 2, buf0, sem0).start()

            dma(2 * i + 1, buf1, sem1).wait()
            carry = chunk(buf1, carry)

            @pl.when(i < NCHUNK // 2 - 1)
            def _():
                dma(2 * i + 3, buf1, sem1).start()

            return carry

        return lax.fori_loop(0, NCHUNK // 2, step, carry)

    def find_pivot(nb, kk):
        nch = nb // L

        def scan(j, carry):
            above, chosen, s_chunk = carry
            c = nch - 1 - j
            acc = hist[pl.ds(c * L, L)]
            sc_ = jnp.sum(acc)
            hit = jnp.logical_and(above < kk, above + sc_ >= kk)
            chosen = jnp.where(hit, c, chosen)
            s_chunk = jnp.where(hit, above, s_chunk)
            return above + sc_, chosen, s_chunk
        _, chosen, s_chunk = lax.fori_loop(
            0, nch, scan, (jnp.int32(0), jnp.int32(0), jnp.int32(0)))

        t16 = hist[pl.ds(chosen * L, L)]
        pref = plsc.cumsum(t16)
        tot = jnp.sum(t16)
        s_j = s_chunk + (tot - pref)
        hit = jnp.logical_and(s_j < kk, s_j + t16 >= kk)
        j_local = jnp.sum(jnp.where(hit, lane, zero16i))
        s_level = jnp.sum(jnp.where(hit, s_j, zero16i))
        c_level = jnp.sum(jnp.where(hit, t16, zero16i))
        return chosen * L + j_local, s_level, c_level

    def inv_key(keyv):
        neg = keyv < jnp.uint32(0x80000000)
        bits = jnp.where(neg, ~keyv, keyv ^ jnp.uint32(0x80000000))
        return plsc.bitcast(bits, jnp.float32)

    def row_body(i, _):
        row = wid * ROWS_PER_W + i
        base = row * R

        clear_hist()

        def vec1(xv, carry):
            key = monotone_key(xv)
            b = plsc.bitcast(key >> jnp.uint32(21), jnp.int32)
            plsc.addupdate_scatter(hist, [b], ones16i)
            return carry
        stream_pass(base, vec1, jnp.int32(0))
        b1, s1, c1 = find_pivot(NB1, jnp.int32(K))
        b1u = b1.astype(jnp.uint32)
        k2 = jnp.int32(K) - s1

        def vec2(xv, carry):
            acc, offv = carry
            key = monotone_key(xv)
            hi = key >> jnp.uint32(21)
            inb = hi == b1u
            maski = jnp.where(inb, ones16i, zero16i)
            pos = offv + (plsc.cumsum(maski) - maski)
            plsc.store_scatter(cand, [jnp.minimum(pos, CAP)], xv, mask=inb)
            npop = plsc.all_reduce_population_count(inb)
            return acc + jnp.where(hi > b1u, xv, zero16f), offv + npop
        acc1, _ = stream_pass(base, vec2, (zero16f, zero16i))
        sum_above1 = jnp.sum(acc1)

        pad_off = jnp.minimum(c1, jnp.int32(CAP))
        padv = plsc.bitcast(jnp.full((L,), -1, jnp.int32), jnp.float32)
        plsc.store_scatter(cand, [pad_off + lane], padv)
        nloc = (c1 + jnp.int32(L - 1)) // jnp.int32(L)
        in_cap = c1 <= jnp.int32(CAP)

        clear_hist()

        def level2_local(_):
            def body(j, acc):
                xv = cand[pl.ds(j * L, L)]
                key = monotone_key(xv)
                b = plsc.bitcast((key >> jnp.uint32(10)) & jnp.uint32(0x7FF),
                                 jnp.int32)
                plsc.addupdate_scatter(hist, [b], ones16i)
                return acc
            return lax.fori_loop(0, nloc, body, jnp.int32(0))

        def level2_stream(_):
            def vec2h(xv, acc):
                key = monotone_key(xv)
                hi = key >> jnp.uint32(21)
                b = plsc.bitcast((key >> jnp.uint32(10)) & jnp.uint32(0x7FF),
                                 jnp.int32)
                plsc.addupdate_scatter(hist, [b], ones16i,
                                       mask=hi == b1u)
                return acc
            return stream_pass(base, vec2h, jnp.int32(0))

        lax.cond(in_cap, level2_local, level2_stream, jnp.int32(0))
        b2, s2, _ = find_pivot(NB2, k2)
        k3 = k2 - s2
        pref2 = (b1u << jnp.uint32(11)) | b2.astype(jnp.uint32)

        clear_hist()
        top1 = (b1u << jnp.uint32(11)) | jnp.uint32(0x7FF)

        def level3_local(_):
            def body(j, acc):
                xv = cand[pl.ds(j * L, L)]
                key = monotone_key(xv)
                kp = key >> jnp.uint32(10)
                b = plsc.bitcast(key & jnp.uint32(0x3FF), jnp.int32)
                plsc.addupdate_scatter(hist, [b], ones16i,
                                       mask=kp == pref2)
                return acc + jnp.where(kp > pref2, xv, zero16f)
            acc2 = lax.fori_loop(0, nloc, body, zero16f)
            return jnp.sum(acc2)

        def level3_stream(_):
            def vec3(xv, acc):
                key = monotone_key(xv)
                kp = key >> jnp.uint32(10)
                b = plsc.bitcast(key & jnp.uint32(0x3FF), jnp.int32)
                plsc.addupdate_scatter(hist, [b], ones16i,
                                       mask=kp == pref2)
                mid = jnp.logical_and(kp > pref2, kp <= top1)
                return acc + jnp.where(mid, xv, zero16f)
            acc2 = stream_pass(base, vec3, zero16f)
            return jnp.sum(acc2)

        sum_above2 = lax.cond(in_cap, level3_local, level3_stream,
                              jnp.int32(0))
        b3, s3, _ = find_pivot(NB3, k3)

        pref_full = pref2 << jnp.uint32(10)

        def rec(c, acc):
            j = c * L + lane
            cnt = hist[pl.ds(c * L, L)]
            keyv = pref_full | plsc.bitcast(j, jnp.uint32)
            val = inv_key(keyv)
            sel = jnp.where(j > b3, cnt.astype(jnp.float32) * val, zero16f)
            return acc + sel
        acc3 = lax.fori_loop(0, NB3 // L, rec, zero16f)
        sum3 = jnp.sum(acc3)

        tvec = inv_key(jnp.broadcast_to(pref_full, (L,))
                       | plsc.bitcast(jnp.broadcast_to(b3, (L,)), jnp.uint32))
        n_rem = (jnp.int32(K) - (s1 + s2 + s3)).astype(jnp.float32)
        tie = jnp.sum(jnp.where(lane == 0, n_rem * tvec, zero16f))

        mean = (sum_above1 + sum_above2 + sum3 + tie) * jnp.float32(1.0 / K)
        res[...] = jnp.broadcast_to(mean, (L,))
        pltpu.sync_copy(res, out_hbm.at[row])
        return 0

    lax.fori_loop(0, ROWS_PER_W, row_body, 0)


@jax.jit
def _sc_topk_mean(xf):
    mesh = plsc.VectorSubcoreMesh(core_axis_name="c", subcore_axis_name="s")
    fn = pl.kernel(
        _sc_body,
        out_type=jax.ShapeDtypeStruct((NROWS, L), jnp.float32),
        mesh=mesh,
        compiler_params=pltpu.CompilerParams(needs_layout_passes=False),
        scratch_types=[
            pltpu.VMEM((HIST_WORDS,), jnp.int32),
            pltpu.VMEM((CH,), jnp.float32),
            pltpu.VMEM((CH,), jnp.float32),
            pltpu.VMEM((CAP + L,), jnp.float32),
            pltpu.VMEM((L,), jnp.float32),
            pltpu.SemaphoreType.DMA,
            pltpu.SemaphoreType.DMA,
        ],
    )
    return fn(xf)


def kernel(x):
    b, h, w, c = x.shape
    xf = x.reshape(-1)
    out = _sc_topk_mean(xf)
    return out[:, 0].reshape(b, c)


# speedup vs baseline: 1.0204x; 1.0204x over previous
import functools

import jax
import jax.numpy as jnp
from jax import lax
from jax.experimental import pallas as pl
from jax.experimental.pallas import tpu as pltpu
from jax.experimental.pallas import tpu_sc as plsc

NC, NS, L = 2, 16, 16
NW = NC * NS
R = 147456
NROWS = 384
ROWS_PER_W = NROWS // NW
K = 2949
CH = 12288
NCHUNK = R // CH
UNROLL = 8
NB1, NB2, NB3 = 2048, 2048, 1024
HIST_WORDS = NB1
CAP = 32768


def _sc_body(x_hbm, out_hbm, hist, buf0, buf1, cand, res, sem0, sem1):
    wid = lax.axis_index("s") * NC + lax.axis_index("c")
    lane = lax.iota(jnp.int32, L)
    lane_u = lax.iota(jnp.uint32, L)
    zero16i = jnp.zeros((L,), jnp.int32)
    zero16f = jnp.zeros((L,), jnp.float32)
    ones16i = jnp.full((L,), 1, jnp.int32)

    def monotone_key(xv):
        u = plsc.bitcast(xv, jnp.uint32)
        s = plsc.bitcast(
            lax.shift_right_arithmetic(plsc.bitcast(xv, jnp.int32), 31),
            jnp.uint32)
        return u ^ (s | jnp.uint32(0x80000000))

    def clear_hist():
        @plsc.parallel_loop(0, HIST_WORDS // L, unroll=8)
        def _(i):
            hist[pl.ds(i * L, L)] = zero16i

    def stream_pass(base, vec_fn, carry):
        def dma(c, buf, sem):
            return pltpu.make_async_copy(
                x_hbm.at[pl.ds(base + c * CH, CH)], buf, sem)

        dma(0, buf0, sem0).start()
        dma(1, buf1, sem1).start()

        def chunk(buf, carry):
            @plsc.parallel_loop(0, CH // L, unroll=UNROLL, carry=carry)
            def out(j, c):
                xv = buf[pl.ds(j * L, L)]
                return vec_fn(xv, c)
            return out

        def step(i, carry):
            dma(2 * i, buf0, sem0).wait()
            carry = chunk(buf0, carry)

            @pl.when(i < NCHUNK // 2 - 1)
            def _():
                dma(2 * i + 2, buf0, sem0).start()

            dma(2 * i + 1, buf1, sem1).wait()
            carry = chunk(buf1, carry)

            @pl.when(i < NCHUNK // 2 - 1)
            def _():
                dma(2 * i + 3, buf1, sem1).start()

            return carry

        return lax.fori_loop(0, NCHUNK // 2, step, carry)

    def find_pivot(nb, kk):
        nch = nb // L
        nsup = nch // L

        above = jnp.int32(0)
        chosen_s = jnp.int32(0)
        s_sup = jnp.int32(0)
        for j in range(nsup - 1, -1, -1):
            acc = hist[pl.ds(j * (L * L), L)]
            for t in range(1, L):
                acc = acc + hist[pl.ds(j * (L * L) + t * L, L)]
            ss = jnp.sum(acc)
            hit = jnp.logical_and(above < kk, above + ss >= kk)
            chosen_s = jnp.where(hit, jnp.int32(j), chosen_s)
            s_sup = jnp.where(hit, above, s_sup)
            above = above + ss

        def scan(j, carry):
            above, chosen, s_chunk = carry
            c = chosen_s * L + (L - 1 - j)
            acc = hist[pl.ds(c * L, L)]
            sc_ = jnp.sum(acc)
            hit = jnp.logical_and(above < kk, above + sc_ >= kk)
            chosen = jnp.where(hit, c, chosen)
            s_chunk = jnp.where(hit, above, s_chunk)
            return above + sc_, chosen, s_chunk
        _, chosen, s_chunk = lax.fori_loop(
            0, L, scan, (s_sup, jnp.int32(0), jnp.int32(0)))

        t16 = hist[pl.ds(chosen * L, L)]
        pref = plsc.cumsum(t16)
        tot = jnp.sum(t16)
        s_j = s_chunk + (tot - pref)
        hit = jnp.logical_and(s_j < kk, s_j + t16 >= kk)
        j_local = jnp.sum(jnp.where(hit, lane, zero16i))
        s_level = jnp.sum(jnp.where(hit, s_j, zero16i))
        c_level = jnp.sum(jnp.where(hit, t16, zero16i))
        return chosen * L + j_local, s_level, c_level

    def inv_key(keyv):
        neg = keyv < jnp.uint32(0x80000000)
        bits = jnp.where(neg, ~keyv, keyv ^ jnp.uint32(0x80000000))
        return plsc.bitcast(bits, jnp.float32)

    def row_body(i, _):
        row = wid * ROWS_PER_W + i
        base = row * R

        clear_hist()

        def vec1(xv, carry):
            key = monotone_key(xv)
            b = plsc.bitcast(key >> jnp.uint32(21), jnp.int32)
            plsc.addupdate_scatter(hist, [b], ones16i)
            return carry
        stream_pass(base, vec1, jnp.int32(0))
        b1, s1, c1 = find_pivot(NB1, jnp.int32(K))
        b1u = b1.astype(jnp.uint32)
        k2 = jnp.int32(K) - s1

        def vec2(xv, carry):
            acc, offv = carry
            key = monotone_key(xv)
            hi = key >> jnp.uint32(21)
            inb = hi == b1u
            maski = jnp.where(inb, ones16i, zero16i)
            pos = offv + (plsc.cumsum(maski) - maski)
            plsc.store_scatter(cand, [jnp.minimum(pos, CAP)], xv, mask=inb)
            npop = plsc.all_reduce_population_count(inb)
            return acc + jnp.where(hi > b1u, xv, zero16f), offv + npop
        acc1, _ = stream_pass(base, vec2, (zero16f, zero16i))
        sum_above1 = jnp.sum(acc1)

        pad_off = jnp.minimum(c1, jnp.int32(CAP))
        padv = plsc.bitcast(jnp.full((L,), -1, jnp.int32), jnp.float32)
        plsc.store_scatter(cand, [pad_off + lane], padv)
        nloc = (c1 + jnp.int32(L - 1)) // jnp.int32(L)
        in_cap = c1 <= jnp.int32(CAP)

        clear_hist()

        def level2_local(_):
            def body(j, acc):
                xv = cand[pl.ds(j * L, L)]
                key = monotone_key(xv)
                b = plsc.bitcast((key >> jnp.uint32(10)) & jnp.uint32(0x7FF),
                                 jnp.int32)
                plsc.addupdate_scatter(hist, [b], ones16i)
                return acc
            return lax.fori_loop(0, nloc, body, jnp.int32(0))

        def level2_stream(_):
            def vec2h(xv, acc):
                key = monotone_key(xv)
                hi = key >> jnp.uint32(21)
                b = plsc.bitcast((key >> jnp.uint32(10)) & jnp.uint32(0x7FF),
                                 jnp.int32)
                plsc.addupdate_scatter(hist, [b], ones16i,
                                       mask=hi == b1u)
                return acc
            return stream_pass(base, vec2h, jnp.int32(0))

        lax.cond(in_cap, level2_local, level2_stream, jnp.int32(0))
        b2, s2, _ = find_pivot(NB2, k2)
        k3 = k2 - s2
        pref2 = (b1u << jnp.uint32(11)) | b2.astype(jnp.uint32)

        clear_hist()
        top1 = (b1u << jnp.uint32(11)) | jnp.uint32(0x7FF)

        def level3_local(_):
            def body(j, acc):
                xv = cand[pl.ds(j * L, L)]
                key = monotone_key(xv)
                kp = key >> jnp.uint32(10)
                b = plsc.bitcast(key & jnp.uint32(0x3FF), jnp.int32)
                plsc.addupdate_scatter(hist, [b], ones16i,
                                       mask=kp == pref2)
                return acc + jnp.where(kp > pref2, xv, zero16f)
            acc2 = lax.fori_loop(0, nloc, body, zero16f)
            return jnp.sum(acc2)

        def level3_stream(_):
            def vec3(xv, acc):
                key = monotone_key(xv)
                kp = key >> jnp.uint32(10)
                b = plsc.bitcast(key & jnp.uint32(0x3FF), jnp.int32)
                plsc.addupdate_scatter(hist, [b], ones16i,
                                       mask=kp == pref2)
                mid = jnp.logical_and(kp > pref2, kp <= top1)
                return acc + jnp.where(mid, xv, zero16f)
            acc2 = stream_pass(base, vec3, zero16f)
            return jnp.sum(acc2)

        sum_above2 = lax.cond(in_cap, level3_local, level3_stream,
                              jnp.int32(0))
        b3, s3, _ = find_pivot(NB3, k3)

        pref_full = pref2 << jnp.uint32(10)

        def rec(c, acc):
            j = c * L + lane
            cnt = hist[pl.ds(c * L, L)]
            keyv = pref_full | plsc.bitcast(j, jnp.uint32)
            val = inv_key(keyv)
            sel = jnp.where(j > b3, cnt.astype(jnp.float32) * val, zero16f)
            return acc + sel
        acc3 = lax.fori_loop(0, NB3 // L, rec, zero16f)
        sum3 = jnp.sum(acc3)

        tvec = inv_key(jnp.broadcast_to(pref_full, (L,))
                       | plsc.bitcast(jnp.broadcast_to(b3, (L,)), jnp.uint32))
        n_rem = (jnp.int32(K) - (s1 + s2 + s3)).astype(jnp.float32)
        tie = jnp.sum(jnp.where(lane == 0, n_rem * tvec, zero16f))

        mean = (sum_above1 + sum_above2 + sum3 + tie) * jnp.float32(1.0 / K)
        res[...] = jnp.broadcast_to(mean, (L,))
        pltpu.sync_copy(res, out_hbm.at[row])
        return 0

    lax.fori_loop(0, ROWS_PER_W, row_body, 0)


@jax.jit
def _sc_topk_mean(xf):
    mesh = plsc.VectorSubcoreMesh(core_axis_name="c", subcore_axis_name="s")
    fn = pl.kernel(
        _sc_body,
        out_type=jax.ShapeDtypeStruct((NROWS, L), jnp.float32),
        mesh=mesh,
        compiler_params=pltpu.CompilerParams(needs_layout_passes=False),
        scratch_types=[
            pltpu.VMEM((HIST_WORDS,), jnp.int32),
            pltpu.VMEM((CH,), jnp.float32),
            pltpu.VMEM((CH,), jnp.float32),
            pltpu.VMEM((CAP + L,), jnp.float32),
            pltpu.VMEM((L,), jnp.float32),
            pltpu.SemaphoreType.DMA,
            pltpu.SemaphoreType.DMA,
        ],
    )
    return fn(xf)


def kernel(x):
    b, h, w, c = x.shape
    xf = x.reshape(-1)
    out = _sc_topk_mean(xf)
    return out[:, 0].reshape(b, c)
